# baseline (speedup 1.0000x reference)
import functools

import jax
import jax.numpy as jnp
from jax import lax
from jax.experimental import pallas as pl
from jax.experimental.pallas import tpu as pltpu
from jax.experimental.pallas import tpu_sc as plsc

NC = 2
NS = 16
NW = NC * NS
CH = 128
NCH0, NCH1 = 40, 40


def _make_aggregate(n_pad, F, nch0, nch1, R, ch):
  n_max = max(nch0, nch1)
  rpt = n_pad // NS
  assert nch0 % R == 0 and nch1 % R == 0

  mesh = plsc.VectorSubcoreMesh(
      core_axis_name="c", subcore_axis_name="s",
      num_cores=NC, num_subcores=NS)

  @functools.partial(
      pl.kernel,
      out_type=jax.ShapeDtypeStruct((NC, n_pad, F), jnp.float32),
      mesh=mesh,
      compiler_params=pltpu.CompilerParams(use_tc_tiling_on_sc=False),
      scratch_types=[
          pltpu.VMEM((n_max, ch), jnp.int32),
          pltpu.VMEM((n_max, ch), jnp.int32),
          pltpu.VMEM((R, ch, F), jnp.float32),
          pltpu.VMEM_SHARED((n_pad, F), jnp.float32),
          [pltpu.SemaphoreType.DMA] * R,
      ],
  )
  def agg_kernel(table_hbm, src_hbm, dst_hbm, zeros_hbm, out_hbm,
                 src_v, dst_v, rows_v, acc_sh, gsems):
    cid = lax.axis_index("c")
    sid = lax.axis_index("s")
    wid = sid * NC + cid
    r0 = sid * rpt
    pltpu.sync_copy(src_hbm.at[wid], src_v)
    pltpu.sync_copy(dst_hbm.at[wid], dst_v)
    pltpu.sync_copy(zeros_hbm.at[pl.ds(r0, rpt)], acc_sh.at[pl.ds(r0, rpt)])
    plsc.subcore_barrier()

    def run(n_ch):
      for b in range(R):
        pltpu.async_copy(table_hbm.at[src_v.at[b]], rows_v.at[b], gsems[b])

      def grp(g, carry):
        i0 = g * R
        for b in range(R):
          i = i0 + b
          pltpu.make_async_copy(
              table_hbm.at[src_v.at[i]], rows_v.at[b], gsems[b]).wait()
          pltpu.sync_copy(rows_v.at[b], acc_sh.at[dst_v.at[i]], add=True)
          nxt = i + R

          @pl.when(nxt < n_ch)
          def _():
            pltpu.async_copy(table_hbm.at[src_v.at[nxt]], rows_v.at[b],
                             gsems[b])
        return carry

      lax.fori_loop(0, n_ch // R, grp, 0)

    if nch0 == nch1:
      run(nch0)
    else:
      @pl.when(cid == 0)
      def _():
        run(nch0)

      @pl.when(cid != 0)
      def _():
        run(nch1)

    plsc.subcore_barrier()
    pltpu.sync_copy(acc_sh.at[pl.ds(r0, rpt)], out_hbm.at[cid, pl.ds(r0, rpt)])

  return agg_kernel


def _make_aggregate_bf16(n_pad, F, nch, R, ch):
  rpt = n_pad // NS
  assert nch % R == 0 and F % 32 == 0
  Fw = F // 2

  mesh = plsc.VectorSubcoreMesh(
      core_axis_name="c", subcore_axis_name="s",
      num_cores=NC, num_subcores=NS)

  @functools.partial(
      pl.kernel,
      out_type=jax.ShapeDtypeStruct((NC, n_pad, F), jnp.float32),
      mesh=mesh,
      compiler_params=pltpu.CompilerParams(use_tc_tiling_on_sc=False,
                                           needs_layout_passes=False),
      scratch_types=[
          pltpu.VMEM((nch, ch), jnp.int32),
          pltpu.VMEM((nch, ch), jnp.int32),
          pltpu.VMEM((R, ch, Fw), jnp.int32),
          pltpu.VMEM((R, ch, F), jnp.float32),
          pltpu.VMEM_SHARED((n_pad, F), jnp.float32),
          [pltpu.SemaphoreType.DMA] * R,
      ],
  )
  def agg_kernel(table_hbm, src_hbm, dst_hbm, out_hbm,
                 src_v, dst_v, raw_v, rows_v, acc_sh, gsems):
    cid = lax.axis_index("c")
    sid = lax.axis_index("s")
    wid = sid * NC + cid
    r0 = sid * rpt
    pltpu.sync_copy(src_hbm.at[wid], src_v)
    pltpu.sync_copy(dst_hbm.at[wid], dst_v)

    def zrow(r, c):
      for k in range(F // 16):
        rows_v[0, r, pl.ds(16 * k, 16)] = jnp.zeros((16,), jnp.float32)
      return c

    lax.fori_loop(0, ch, zrow, 0)
    for t in range(rpt // ch):
      pltpu.sync_copy(rows_v.at[0], acc_sh.at[pl.ds(r0 + t * ch, ch)])
    plsc.subcore_barrier()

    for b in range(R):
      pltpu.async_copy(table_hbm.at[src_v.at[b]], raw_v.at[b], gsems[b])

    def widen_row(r, b):
      for k in range(F // 32):
        w = raw_v[b, r, pl.ds(16 * k, 16)]
        rows_v[b, r, pl.ds(32 * k, 16)] = plsc.bitcast(
            lax.shift_left(w, 16), jnp.float32)
        rows_v[b, r, pl.ds(32 * k + 16, 16)] = plsc.bitcast(
            lax.bitwise_and(w, jnp.int32(-65536)), jnp.float32)

    def grp(g, carry):
      i0 = g * R
      for b in range(R):
        i = i0 + b
        pltpu.make_async_copy(
            table_hbm.at[src_v.at[i]], raw_v.at[b], gsems[b]).wait()

        def row_body(r, c):
          widen_row(r, b)
          return c

        lax.fori_loop(0, ch, row_body, 0)
        nxt = i + R

        @pl.when(nxt < nch)
        def _():
          pltpu.async_copy(table_hbm.at[src_v.at[nxt]], raw_v.at[b],
                           gsems[b])
        pltpu.sync_copy(rows_v.at[b], acc_sh.at[dst_v.at[i]], add=True)
      return carry

    lax.fori_loop(0, nch // R, grp, 0)
    plsc.subcore_barrier()
    pltpu.sync_copy(acc_sh.at[pl.ds(r0, rpt)], out_hbm.at[cid, pl.ds(r0, rpt)])

  return agg_kernel


def _perm_cols(W):
  r, F = W.shape
  V = W.reshape(r, F // 32, 16, 2)
  return jnp.concatenate([V[..., 0], V[..., 1]], axis=2).reshape(r, F)


def _perm_rows(W):
  F = W.shape[0]
  V = W.reshape(F // 32, 16, 2, -1)
  return jnp.concatenate([V[:, :, 0, :], V[:, :, 1, :]],
                         axis=1).reshape(F, -1)


def _tile_indices(flat, fill, nch0, nch1, ch):
  n_max = max(nch0, nch1)
  n0 = NS * nch0 * ch
  parts = []
  for cnt, blk in ((nch0, flat[:n0]), (nch1, flat[n0:])):
    t = blk.reshape(NS, cnt, ch)
    if cnt < n_max:
      t = jnp.concatenate(
          [t, jnp.full((NS, n_max - cnt, ch), fill, jnp.int32)], axis=1)
    parts.append(t)
  return jnp.stack(parts, axis=1).reshape(NW, n_max, ch)


def _conv_body(x_ref, w_ref, b_ref, o_ref):
  acc = jnp.dot(x_ref[...], w_ref[...], preferred_element_type=jnp.float32)
  o_ref[...] = jnp.maximum(acc + b_ref[...], 0.0).astype(jnp.bfloat16)


def _mid_body(h0_ref, a0_ref, a1_ref, ws1_ref, wn1_ref, b1_ref, wn2_ref,
              h1_ref, p2_ref):
  a = a0_ref[0] + a1_ref[0]
  hn = a / jnp.maximum(a[:, 31:32], 1.0)
  h1 = jnp.maximum(
      jnp.dot(h0_ref[...].astype(jnp.float32), ws1_ref[...],
              preferred_element_type=jnp.float32)
      + jnp.dot(hn, wn1_ref[...], preferred_element_type=jnp.float32)
      + b1_ref[...], 0.0)
  h1_ref[...] = h1
  p2_ref[...] = jnp.dot(
      h1, wn2_ref[...],
      preferred_element_type=jnp.float32).astype(jnp.bfloat16)


def _head_body(h1_ref, a0_ref, a1_ref, c0_ref, c1_ref, ws2_ref, b2_ref,
               wf1_ref, bf1_ref, wf2_ref, bf2_ref, wf3_ref, bf3_ref, o_ref):
  deg = jnp.maximum(a0_ref[0, :, 31:32] + a1_ref[0, :, 31:32], 1.0)
  n2 = (c0_ref[0] + c1_ref[0]) / deg
  h2 = jnp.maximum(
      jnp.dot(h1_ref[...], ws2_ref[...], preferred_element_type=jnp.float32)
      + n2 + b2_ref[...], 0.0)
  t = jnp.maximum(
      jnp.dot(h2, wf1_ref[...], preferred_element_type=jnp.float32)
      + bf1_ref[...], 0.0)
  t = jnp.maximum(
      jnp.dot(t, wf2_ref[...], preferred_element_type=jnp.float32)
      + bf2_ref[...], 0.0)
  o_ref[...] = (jnp.dot(t, wf3_ref[...], preferred_element_type=jnp.float32)
                + bf3_ref[...])


def _full(shape):
  nd = len(shape)
  return pl.BlockSpec(shape, lambda i: (0,) * nd)


def kernel(x, edge_index, conv1d_w, conv1d_b, W_self1, W_neigh1, b1,
           W_self2, W_neigh2, b2, W_fc1, b_fc1, W_fc2, b_fc2, W_fc3, b_fc3):
  N, D = x.shape
  E = edge_index.shape[1]
  K = conv1d_w.shape[2]
  S = 8
  C1 = W_self1.shape[0]
  C1P = C1 + 1
  H = W_self1.shape[1]
  H2 = W_self2.shape[1]

  BLK = 1024
  n_pad = ((N + 1 + BLK - 1) // BLK) * BLK
  e_pad = NS * (NCH0 + NCH1) * CH
  assert e_pad >= E
  grid = (n_pad // BLK,)

  f32 = jnp.float32

  rows = (jnp.arange(C1)[:, None] * S + jnp.arange(K)[None, :]).reshape(-1)
  cols = jnp.repeat(jnp.arange(C1), K)
  Wc = jnp.zeros((D, C1P), f32).at[rows, cols].set(
      jnp.tile(conv1d_w[0, 0], C1))
  bc = jnp.concatenate([jnp.broadcast_to(conv1d_b, (C1,)),
                        jnp.ones((1,), f32)]).reshape(1, C1P)

  Ws1p = jnp.zeros((C1P, H), f32).at[:C1].set(W_self1)
  Wn1p = jnp.zeros((C1P, H), f32).at[:C1].set(W_neigh1)

  x_pad = jnp.zeros((n_pad, D), f32).at[:N].set(x)
  src_flat = jnp.zeros((e_pad,), jnp.int32).at[:E].set(edge_index[0])
  dst_flat = jnp.full((e_pad,), N, jnp.int32).at[:E].set(edge_index[1])

  h0p = pl.pallas_call(
      _conv_body,
      grid=grid,
      in_specs=[pl.BlockSpec((BLK, D), lambda i: (i, 0)),
                _full((D, C1P)), _full((1, C1P))],
      out_specs=pl.BlockSpec((BLK, C1P), lambda i: (i, 0)),
      out_shape=jax.ShapeDtypeStruct((n_pad, C1P), jnp.bfloat16),
  )(x_pad, Wc, bc)

  src_t = _tile_indices(src_flat, 0, NCH0, NCH1, CH)
  dst_t = _tile_indices(dst_flat, N, NCH0, NCH1, CH)
  h0w = lax.bitcast_convert_type(h0p.reshape(n_pad, C1P // 2, 2), jnp.int32)
  agg1 = _make_aggregate_bf16(n_pad, C1P, NCH0, 2, CH)(
      h0w, src_t, dst_t)

  h1, p2 = pl.pallas_call(
      _mid_body,
      grid=grid,
      in_specs=[pl.BlockSpec((BLK, C1P), lambda i: (i, 0)),
                pl.BlockSpec((1, BLK, C1P), lambda i: (0, i, 0)),
                pl.BlockSpec((1, BLK, C1P), lambda i: (1, i, 0)),
                _full((C1P, H)), _full((C1P, H)), _full((1, H)),
                _full((H, H2))],
      out_specs=[pl.BlockSpec((BLK, H), lambda i: (i, 0)),
                 pl.BlockSpec((BLK, H2), lambda i: (i, 0))],
      out_shape=[jax.ShapeDtypeStruct((n_pad, H), f32),
                 jax.ShapeDtypeStruct((n_pad, H2), jnp.bfloat16)],
  )(h0p, agg1, agg1, Ws1p, _perm_rows(Wn1p), b1.reshape(1, H), W_neigh2)

  p2w = lax.bitcast_convert_type(p2.reshape(n_pad, H2 // 2, 2), jnp.int32)
  ch2 = 64
  nch2 = e_pad // (NW * ch2)
  agg2 = _make_aggregate_bf16(n_pad, H2, nch2, 2, ch2)(
      p2w, src_t.reshape(NW, nch2, ch2), dst_t.reshape(NW, nch2, ch2))

  H4 = W_fc1.shape[1]
  H8 = W_fc2.shape[1]
  out = pl.pallas_call(
      _head_body,
      grid=grid,
      in_specs=[pl.BlockSpec((BLK, H), lambda i: (i, 0)),
                pl.BlockSpec((1, BLK, C1P), lambda i: (0, i, 0)),
                pl.BlockSpec((1, BLK, C1P), lambda i: (1, i, 0)),
                pl.BlockSpec((1, BLK, H2), lambda i: (0, i, 0)),
                pl.BlockSpec((1, BLK, H2), lambda i: (1, i, 0)),
                _full((H, H2)), _full((1, H2)),
                _full((H2, H4)), _full((1, H4)),
                _full((H4, H8)), _full((1, H8)),
                _full((H8, 1)), _full((1, 1))],
      out_specs=pl.BlockSpec((BLK, 1), lambda i: (i, 0)),
      out_shape=jax.ShapeDtypeStruct((n_pad, 1), f32),
  )(h1, agg1, agg1, agg2, agg2, _perm_cols(W_self2),
    _perm_rows(b2[:, None]).reshape(1, H2), _perm_rows(W_fc1),
    b_fc1.reshape(1, H4), W_fc2, b_fc2.reshape(1, H8),
    W_fc3, b_fc3.reshape(1, 1))

  return out[:N]


# speedup vs baseline: 1.0401x; 1.0401x over previous
import functools

import jax
import jax.numpy as jnp
from jax import lax
from jax.experimental import pallas as pl
from jax.experimental.pallas import tpu as pltpu
from jax.experimental.pallas import tpu_sc as plsc

NC = 2
NS = 16
NW = NC * NS
CH = 128
NCH = 40


def _make_aggregate_bf16(n_pad, F, nch, R, ch):
  rpt = n_pad // NS
  assert nch % R == 0 and F % 32 == 0
  Fw = F // 2

  mesh = plsc.VectorSubcoreMesh(
      core_axis_name="c", subcore_axis_name="s",
      num_cores=NC, num_subcores=NS)

  @functools.partial(
      pl.kernel,
      out_type=jax.ShapeDtypeStruct((NC, n_pad, F), jnp.float32),
      mesh=mesh,
      compiler_params=pltpu.CompilerParams(use_tc_tiling_on_sc=False,
                                           needs_layout_passes=False),
      scratch_types=[
          pltpu.VMEM((nch, ch), jnp.int32),
          pltpu.VMEM((nch, ch), jnp.int32),
          pltpu.VMEM((R, ch, Fw), jnp.int32),
          pltpu.VMEM((R, ch, F), jnp.float32),
          pltpu.VMEM_SHARED((n_pad, F), jnp.float32),
          [pltpu.SemaphoreType.DMA] * R,
      ],
  )
  def agg_kernel(table_hbm, src_hbm, dst_hbm, zeros_hbm, out_hbm,
                 src_v, dst_v, raw_v, rows_v, acc_sh, gsems):
    cid = lax.axis_index("c")
    sid = lax.axis_index("s")
    wid = sid * NC + cid
    r0 = sid * rpt
    pltpu.sync_copy(src_hbm.at[wid], src_v)
    pltpu.sync_copy(dst_hbm.at[wid], dst_v)
    pltpu.sync_copy(zeros_hbm.at[pl.ds(r0, rpt)], acc_sh.at[pl.ds(r0, rpt)])
    plsc.subcore_barrier()

    for b in range(R):
      pltpu.async_copy(table_hbm.at[src_v.at[b]], raw_v.at[b], gsems[b])

    def widen_row(r, b):
      for k in range(F // 32):
        w = raw_v[b, r, pl.ds(16 * k, 16)]
        rows_v[b, r, pl.ds(32 * k, 16)] = plsc.bitcast(
            lax.shift_left(w, 16), jnp.float32)
        rows_v[b, r, pl.ds(32 * k + 16, 16)] = plsc.bitcast(
            lax.bitwise_and(w, jnp.int32(-65536)), jnp.float32)

    def grp(g, carry):
      i0 = g * R
      for b in range(R):
        i = i0 + b
        pltpu.make_async_copy(
            table_hbm.at[src_v.at[i]], raw_v.at[b], gsems[b]).wait()

        def row_body(r, c):
          widen_row(r, b)
          return c

        lax.fori_loop(0, ch, row_body, 0)
        nxt = i + R

        @pl.when(nxt < nch)
        def _():
          pltpu.async_copy(table_hbm.at[src_v.at[nxt]], raw_v.at[b],
                           gsems[b])
        pltpu.sync_copy(rows_v.at[b], acc_sh.at[dst_v.at[i]], add=True)
      return carry

    lax.fori_loop(0, nch // R, grp, 0)
    plsc.subcore_barrier()
    pltpu.sync_copy(acc_sh.at[pl.ds(r0, rpt)], out_hbm.at[cid, pl.ds(r0, rpt)])

  return agg_kernel


def _perm_cols(W):
  r, F = W.shape
  V = W.reshape(r, F // 32, 16, 2)
  return jnp.concatenate([V[..., 0], V[..., 1]], axis=2).reshape(r, F)


def _perm_rows(W):
  F = W.shape[0]
  V = W.reshape(F // 32, 16, 2, -1)
  return jnp.concatenate([V[:, :, 0, :], V[:, :, 1, :]],
                         axis=1).reshape(F, -1)


def _tile_indices(flat, fill, nch0, nch1, ch):
  n_max = max(nch0, nch1)
  n0 = NS * nch0 * ch
  parts = []
  for cnt, blk in ((nch0, flat[:n0]), (nch1, flat[n0:])):
    t = blk.reshape(NS, cnt, ch)
    if cnt < n_max:
      t = jnp.concatenate(
          [t, jnp.full((NS, n_max - cnt, ch), fill, jnp.int32)], axis=1)
    parts.append(t)
  return jnp.stack(parts, axis=1).reshape(NW, n_max, ch)


def _conv_body(x_ref, w_ref, b_ref, o_ref):
  acc = jnp.dot(x_ref[...], w_ref[...], preferred_element_type=jnp.float32)
  o_ref[...] = jnp.maximum(acc + b_ref[...], 0.0).astype(jnp.bfloat16)


def _mid_body(h0_ref, a0_ref, a1_ref, ws1_ref, wn1_ref, b1_ref, wn2_ref,
              h1_ref, p2_ref):
  a = a0_ref[0] + a1_ref[0]
  hn = a / jnp.maximum(a[:, 31:32], 1.0)
  h1 = jnp.maximum(
      jnp.dot(h0_ref[...].astype(jnp.float32), ws1_ref[...],
              preferred_element_type=jnp.float32)
      + jnp.dot(hn, wn1_ref[...], preferred_element_type=jnp.float32)
      + b1_ref[...], 0.0)
  h1_ref[...] = h1
  p2_ref[...] = jnp.dot(
      h1, wn2_ref[...],
      preferred_element_type=jnp.float32).astype(jnp.bfloat16)


def _head_body(h1_ref, a0_ref, a1_ref, c0_ref, c1_ref, ws2_ref, b2_ref,
               wf1_ref, bf1_ref, wf2_ref, bf2_ref, wf3_ref, bf3_ref, o_ref):
  deg = jnp.maximum(a0_ref[0, :, 31:32] + a1_ref[0, :, 31:32], 1.0)
  n2 = (c0_ref[0] + c1_ref[0]) / deg
  h2 = jnp.maximum(
      jnp.dot(h1_ref[...], ws2_ref[...], preferred_element_type=jnp.float32)
      + n2 + b2_ref[...], 0.0)
  t = jnp.maximum(
      jnp.dot(h2, wf1_ref[...], preferred_element_type=jnp.float32)
      + bf1_ref[...], 0.0)
  t = jnp.maximum(
      jnp.dot(t, wf2_ref[...], preferred_element_type=jnp.float32)
      + bf2_ref[...], 0.0)
  o_ref[...] = (jnp.dot(t, wf3_ref[...], preferred_element_type=jnp.float32)
                + bf3_ref[...])


def _full(shape):
  nd = len(shape)
  return pl.BlockSpec(shape, lambda i: (0,) * nd)


def kernel(x, edge_index, conv1d_w, conv1d_b, W_self1, W_neigh1, b1,
           W_self2, W_neigh2, b2, W_fc1, b_fc1, W_fc2, b_fc2, W_fc3, b_fc3):
  N, D = x.shape
  E = edge_index.shape[1]
  K = conv1d_w.shape[2]
  S = 8
  C1 = W_self1.shape[0]
  C1P = C1 + 1
  H = W_self1.shape[1]
  H2 = W_self2.shape[1]

  BLK = 1024
  n_pad = ((N + 1 + BLK - 1) // BLK) * BLK
  e_pad = NS * 2 * NCH * CH
  assert e_pad >= E
  grid = (n_pad // BLK,)

  f32 = jnp.float32

  rows = (jnp.arange(C1)[:, None] * S + jnp.arange(K)[None, :]).reshape(-1)
  cols = jnp.repeat(jnp.arange(C1), K)
  Wc = jnp.zeros((D, C1P), f32).at[rows, cols].set(
      jnp.tile(conv1d_w[0, 0], C1))
  bc = jnp.concatenate([jnp.broadcast_to(conv1d_b, (C1,)),
                        jnp.ones((1,), f32)]).reshape(1, C1P)

  Ws1p = jnp.zeros((C1P, H), f32).at[:C1].set(W_self1)
  Wn1p = jnp.zeros((C1P, H), f32).at[:C1].set(W_neigh1)

  x_pad = jnp.zeros((n_pad, D), f32).at[:N].set(x)
  src_flat = jnp.zeros((e_pad,), jnp.int32).at[:E].set(edge_index[0])
  dst_flat = jnp.full((e_pad,), N, jnp.int32).at[:E].set(edge_index[1])

  h0p = pl.pallas_call(
      _conv_body,
      grid=grid,
      in_specs=[pl.BlockSpec((BLK, D), lambda i: (i, 0)),
                _full((D, C1P)), _full((1, C1P))],
      out_specs=pl.BlockSpec((BLK, C1P), lambda i: (i, 0)),
      out_shape=jax.ShapeDtypeStruct((n_pad, C1P), jnp.bfloat16),
  )(x_pad, Wc, bc)

  src_t = _tile_indices(src_flat, 0, NCH, NCH, CH)
  dst_t = _tile_indices(dst_flat, N, NCH, NCH, CH)
  h0w = lax.bitcast_convert_type(h0p.reshape(n_pad, C1P // 2, 2), jnp.int32)
  agg1 = _make_aggregate_bf16(n_pad, C1P, NCH, 2, CH)(
      h0w, src_t, dst_t, jnp.zeros((n_pad, C1P), f32))

  h1, p2 = pl.pallas_call(
      _mid_body,
      grid=grid,
      in_specs=[pl.BlockSpec((BLK, C1P), lambda i: (i, 0)),
                pl.BlockSpec((1, BLK, C1P), lambda i: (0, i, 0)),
                pl.BlockSpec((1, BLK, C1P), lambda i: (1, i, 0)),
                _full((C1P, H)), _full((C1P, H)), _full((1, H)),
                _full((H, H2))],
      out_specs=[pl.BlockSpec((BLK, H), lambda i: (i, 0)),
                 pl.BlockSpec((BLK, H2), lambda i: (i, 0))],
      out_shape=[jax.ShapeDtypeStruct((n_pad, H), f32),
                 jax.ShapeDtypeStruct((n_pad, H2), jnp.bfloat16)],
  )(h0p, agg1, agg1, Ws1p, _perm_rows(Wn1p), b1.reshape(1, H), W_neigh2)

  p2w = lax.bitcast_convert_type(p2.reshape(n_pad, H2 // 2, 2), jnp.int32)
  ch2 = 64
  nch2 = e_pad // (NW * ch2)
  agg2 = _make_aggregate_bf16(n_pad, H2, nch2, 2, ch2)(
      p2w, src_t.reshape(NW, nch2, ch2), dst_t.reshape(NW, nch2, ch2),
      jnp.zeros((n_pad, H2), f32))

  H4 = W_fc1.shape[1]
  H8 = W_fc2.shape[1]
  out = pl.pallas_call(
      _head_body,
      grid=grid,
      in_specs=[pl.BlockSpec((BLK, H), lambda i: (i, 0)),
                pl.BlockSpec((1, BLK, C1P), lambda i: (0, i, 0)),
                pl.BlockSpec((1, BLK, C1P), lambda i: (1, i, 0)),
                pl.BlockSpec((1, BLK, H2), lambda i: (0, i, 0)),
                pl.BlockSpec((1, BLK, H2), lambda i: (1, i, 0)),
                _full((H, H2)), _full((1, H2)),
                _full((H2, H4)), _full((1, H4)),
                _full((H4, H8)), _full((1, H8)),
                _full((H8, 1)), _full((1, 1))],
      out_specs=pl.BlockSpec((BLK, 1), lambda i: (i, 0)),
      out_shape=jax.ShapeDtypeStruct((n_pad, 1), f32),
  )(h1, agg1, agg1, agg2, agg2, _perm_cols(W_self2),
    _perm_rows(b2[:, None]).reshape(1, H2), _perm_rows(W_fc1),
    b_fc1.reshape(1, H4), W_fc2, b_fc2.reshape(1, H8),
    W_fc3, b_fc3.reshape(1, 1))

  return out[:N]
